# Initial kernel scaffold; baseline (speedup 1.0000x reference)
import jax
import jax.numpy as jnp
from jax.experimental import pallas as pl


def kernel(x, hour_W, weekday_W, day_W, month_W):
    raise NotImplementedError("write your pallas kernel here")


# speedup vs baseline: 7.1228x; 7.1228x over previous
import jax
import jax.numpy as jnp
from jax.experimental import pallas as pl


def _tc_body(x_ref, w_ref, o_ref):
    xb = x_ref[...]
    lane = jax.lax.broadcasted_iota(jnp.int32, (xb.shape[0], 28), 1)
    v = lane % 7
    t = lane // 7
    x0 = xb[:, 0:1]
    x1 = xb[:, 1:2]
    x2 = xb[:, 2:3]
    x3 = xb[:, 3:4]
    digit = jnp.where(t == 0, x0, jnp.where(t == 1, x1, jnp.where(t == 2, x2, x3)))
    oh = (digit == v).astype(jnp.float32)
    o_ref[...] = jax.lax.dot_general(
        oh, w_ref[...], (((1,), (0,)), ((), ())),
        precision=jax.lax.Precision.HIGHEST)


def kernel(x, hour_W, weekday_W, day_W, month_W):
    B, L, _ = x.shape
    D = hour_W.shape[1]
    N = B * L
    x = x.astype(jnp.int32)
    W = jnp.concatenate([month_W[:7], day_W[:7], weekday_W[:7], hour_W[:7]], 0)
    xf = x.reshape(N, 4)
    BN = 2048
    out = pl.pallas_call(
        _tc_body,
        grid=(N // BN,),
        in_specs=[pl.BlockSpec((BN, 4), lambda i: (i, 0)),
                  pl.BlockSpec((28, D), lambda i: (0, 0))],
        out_specs=pl.BlockSpec((BN, D), lambda i: (i, 0)),
        out_shape=jax.ShapeDtypeStruct((N, D), jnp.float32),
    )(xf, W)
    return out.reshape(B, L, D)


# speedup vs baseline: 7.5809x; 1.0643x over previous
import functools

import jax
import jax.numpy as jnp
from jax import lax
from jax.experimental import pallas as pl
from jax.experimental.pallas import tpu as pltpu
from jax.experimental.pallas import tpu_sc as plsc

_NC, _NS = 2, 16
_NW = _NC * _NS
_TROWS = 2432


def _table_body(w_ref, t_ref):
    k = lax.broadcasted_iota(jnp.int32, (_TROWS, 28), 0)
    c = lax.broadcasted_iota(jnp.int32, (_TROWS, 28), 1)
    t = c // 7
    v = c % 7
    d0 = k // 343
    d1 = (k // 49) % 7
    d2 = (k // 7) % 7
    d3 = k % 7
    digit = jnp.where(t == 0, d0, jnp.where(t == 1, d1, jnp.where(t == 2, d2, d3)))
    oh = (digit == v).astype(jnp.float32)
    t_ref[...] = lax.dot_general(oh, w_ref[...], (((1,), (0,)), ((), ())),
                                 precision=lax.Precision.HIGHEST)


def _make_sc(N, D, E):
    NT = N // _NW
    CH = NT // E
    G = E // 128
    mesh = plsc.VectorSubcoreMesh(core_axis_name="c", subcore_axis_name="s",
                                  num_cores=_NC, num_subcores=_NS)

    @functools.partial(
        pl.kernel,
        out_type=jax.ShapeDtypeStruct((N, D), jnp.float32),
        mesh=mesh,
        compiler_params=pltpu.CompilerParams(needs_layout_passes=False,
                                             use_tc_tiling_on_sc=False),
        scratch_types=[
            pltpu.VMEM((4 * E,), jnp.int32),
            pltpu.VMEM((G, 128), jnp.int32),
            pltpu.VMEM((E, D), jnp.float32),
            pltpu.SemaphoreType.DMA,
        ],
    )
    def sc_main(x_hbm, t_hbm, out_hbm, x_v, keys_v, rows_v, sem):
        wid = lax.axis_index("s") * _NC + lax.axis_index("c")
        lanes = lax.iota(jnp.int32, 16)

        def chunk(it, carry):
            base = wid * NT + it * E
            pltpu.sync_copy(x_hbm.at[pl.ds(base * 4, E * 4)], x_v)

            def keys(g, carry2):
                p = (g * 16 + lanes) * 4
                x0 = plsc.load_gather(x_v, [p])
                x1 = plsc.load_gather(x_v, [p + 1])
                x2 = plsc.load_gather(x_v, [p + 2])
                x3 = plsc.load_gather(x_v, [p + 3])
                key = ((x0 * 7 + x1) * 7 + x2) * 7 + x3
                keys_v[g // 8, pl.ds((g % 8) * 16, 16)] = key
                return carry2

            lax.fori_loop(0, E // 16, keys, 0, unroll=4)

            cps = [pltpu.async_copy(t_hbm.at[keys_v.at[j]],
                                    rows_v.at[pl.ds(j * 128, 128)], sem)
                   for j in range(G)]
            for cp in cps:
                cp.wait()
            pltpu.sync_copy(rows_v, out_hbm.at[pl.ds(base, E)])
            return carry

        lax.fori_loop(0, CH, chunk, 0)

    return sc_main


def kernel(x, hour_W, weekday_W, day_W, month_W):
    B, L, _ = x.shape
    D = hour_W.shape[1]
    N = B * L
    x = x.astype(jnp.int32)
    W = jnp.concatenate([month_W[:7], day_W[:7], weekday_W[:7], hour_W[:7]], 0)
    T = pl.pallas_call(
        _table_body,
        out_shape=jax.ShapeDtypeStruct((_TROWS, D), jnp.float32),
    )(W)
    out = _make_sc(N, D, 2048)(x.reshape(N * 4), T)
    return out.reshape(B, L, D)


# speedup vs baseline: 17.5875x; 2.3200x over previous
import functools

import jax
import jax.numpy as jnp
from jax import lax
from jax.experimental import pallas as pl
from jax.experimental.pallas import tpu as pltpu
from jax.experimental.pallas import tpu_sc as plsc

_NC, _NS = 2, 16
_NW = _NC * _NS
_TR = 608


def _table_body(w_ref, t_ref):
    for q in range(4):
        k = 4 * lax.broadcasted_iota(jnp.int32, (_TR, 28), 0) + q
        c = lax.broadcasted_iota(jnp.int32, (_TR, 28), 1)
        t = c // 7
        v = c % 7
        d0 = k // 343
        d1 = (k // 49) % 7
        d2 = (k // 7) % 7
        d3 = k % 7
        digit = jnp.where(t == 0, d0,
                          jnp.where(t == 1, d1, jnp.where(t == 2, d2, d3)))
        oh = (digit == v).astype(jnp.float32)
        t_ref[:, 32 * q:32 * (q + 1)] = lax.dot_general(
            oh, w_ref[...], (((1,), (0,)), ((), ())),
            precision=lax.Precision.HIGHEST)


def _make_sc(B, L, D):
    BH = B // 128
    NBH = BH // _NW
    DH = D // 8
    mesh = plsc.VectorSubcoreMesh(core_axis_name="c", subcore_axis_name="s",
                                  num_cores=_NC, num_subcores=_NS)

    @functools.partial(
        pl.kernel,
        out_type=jax.ShapeDtypeStruct((L, DH, BH * 8, 128), jnp.float32),
        mesh=mesh,
        compiler_params=pltpu.CompilerParams(needs_layout_passes=False),
        scratch_types=[
            pltpu.VMEM((_TR, 128), jnp.float32),
            pltpu.VMEM((NBH * 4, 128), jnp.int32),
            pltpu.VMEM((DH, NBH * 8, 128), jnp.float32),
        ],
    )
    def sc_main(x_hbm, t_hbm, out_hbm, t_v, x_v, rows_v):
        wid = lax.axis_index("s") * _NC + lax.axis_index("c")
        pltpu.sync_copy(t_hbm, t_v)

        def per_l(l, carry):
            pltpu.sync_copy(x_hbm.at[l, pl.ds(wid * (NBH * 4), NBH * 4)], x_v)

            def group(g, carry2):
                bhi = g // 8
                s = (g % 8) * 16
                x0 = x_v[bhi * 4 + 0, pl.ds(s, 16)]
                x1 = x_v[bhi * 4 + 1, pl.ds(s, 16)]
                x2 = x_v[bhi * 4 + 2, pl.ds(s, 16)]
                x3 = x_v[bhi * 4 + 3, pl.ds(s, 16)]
                key = ((x0 * 7 + x1) * 7 + x2) * 7 + x3
                kq = key >> 2
                kr = (key & 3) << 5
                for d in range(D):
                    vd = plsc.load_gather(t_v, [kq, kr + d])
                    rows_v[d // 8, bhi * 8 + d % 8, pl.ds(s, 16)] = vd
                return carry2

            lax.fori_loop(0, NBH * 8, group, 0)
            pltpu.sync_copy(rows_v,
                            out_hbm.at[l, :, pl.ds(wid * (NBH * 8), NBH * 8)])
            return carry

        lax.fori_loop(0, L, per_l, 0)

    return sc_main


def kernel(x, hour_W, weekday_W, day_W, month_W):
    B, L, _ = x.shape
    D = hour_W.shape[1]
    BH = B // 128
    x = x.astype(jnp.int32)
    W = jnp.concatenate([month_W[:7], day_W[:7], weekday_W[:7], hour_W[:7]], 0)
    T = pl.pallas_call(
        _table_body,
        out_shape=jax.ShapeDtypeStruct((_TR, 128), jnp.float32),
    )(W)
    x3 = (x.transpose(1, 2, 0).reshape(L, 4, BH, 128)
          .transpose(0, 2, 1, 3).reshape(L, BH * 4, 128))
    out4 = _make_sc(B, L, D)(x3, T)
    return (out4.reshape(L, D // 8, BH, 8, 128).transpose(2, 4, 0, 1, 3)
            .reshape(B, L, D))


# speedup vs baseline: 19.5419x; 1.1111x over previous
import functools

import jax
import jax.numpy as jnp
from jax import lax
from jax.experimental import pallas as pl
from jax.experimental.pallas import tpu as pltpu
from jax.experimental.pallas import tpu_sc as plsc

_NC, _NS = 2, 16
_NW = _NC * _NS
_TR = 608


def _table_body(w_ref, t_ref):
    for q in range(4):
        k = 4 * lax.broadcasted_iota(jnp.int32, (_TR, 28), 0) + q
        c = lax.broadcasted_iota(jnp.int32, (_TR, 28), 1)
        t = c // 7
        v = c % 7
        d0 = k // 343
        d1 = (k // 49) % 7
        d2 = (k // 7) % 7
        d3 = k % 7
        digit = jnp.where(t == 0, d0,
                          jnp.where(t == 1, d1, jnp.where(t == 2, d2, d3)))
        oh = (digit == v).astype(jnp.float32)
        t_ref[:, 32 * q:32 * (q + 1)] = lax.dot_general(
            oh, w_ref[...], (((1,), (0,)), ((), ())),
            precision=lax.Precision.HIGHEST)


def _make_sc(B, L, D):
    BH = B // 128
    NBH = BH // _NW
    DH = D // 8
    mesh = plsc.VectorSubcoreMesh(core_axis_name="c", subcore_axis_name="s",
                                  num_cores=_NC, num_subcores=_NS)

    @functools.partial(
        pl.kernel,
        out_type=jax.ShapeDtypeStruct((L, DH, BH * 8, 128), jnp.float32),
        mesh=mesh,
        compiler_params=pltpu.CompilerParams(needs_layout_passes=False),
        scratch_types=[
            pltpu.VMEM((_TR, 128), jnp.float32),
            pltpu.VMEM((2, NBH * 4, 128), jnp.int32),
            pltpu.VMEM((2, DH, NBH * 8, 128), jnp.float32),
            pltpu.SemaphoreType.DMA,
            pltpu.SemaphoreType.DMA,
            pltpu.SemaphoreType.DMA,
            pltpu.SemaphoreType.DMA,
        ],
    )
    def sc_main(x_hbm, t_hbm, out_hbm, t_v, x_v, rows_v, xs0, xs1, os0, os1):
        wid = lax.axis_index("s") * _NC + lax.axis_index("c")
        xcol = wid * (NBH * 4)
        ocol = wid * (NBH * 8)
        pltpu.sync_copy(t_hbm, t_v)

        def x_slice(l):
            return x_hbm.at[l, pl.ds(xcol, NBH * 4)]

        def o_slice(l):
            return out_hbm.at[l, :, pl.ds(ocol, NBH * 8)]

        def compute(xb, rb):
            def group(g, carry2):
                bhi = g // 8
                s = (g % 8) * 16
                x0 = x_v[xb, bhi * 4 + 0, pl.ds(s, 16)]
                x1 = x_v[xb, bhi * 4 + 1, pl.ds(s, 16)]
                x2 = x_v[xb, bhi * 4 + 2, pl.ds(s, 16)]
                x3 = x_v[xb, bhi * 4 + 3, pl.ds(s, 16)]
                key = ((x0 * 7 + x1) * 7 + x2) * 7 + x3
                kq = key >> 2
                kr = (key & 3) << 5
                for d in range(D):
                    vd = plsc.load_gather(t_v, [kq, kr + d])
                    rows_v[rb, d // 8, bhi * 8 + d % 8, pl.ds(s, 16)] = vd
                return carry2

            lax.fori_loop(0, NBH * 8, group, 0, unroll=2)

        pltpu.async_copy(x_slice(0), x_v.at[0], xs0)

        def body2(i, carry):
            l0 = 2 * i
            l1 = l0 + 1
            pltpu.async_copy(x_slice(l1), x_v.at[1], xs1)
            pltpu.make_async_copy(x_slice(l0), x_v.at[0], xs0).wait()

            @pl.when(i >= 1)
            def _():
                pltpu.make_async_copy(rows_v.at[0], o_slice(l0 - 2), os0).wait()

            compute(0, 0)
            pltpu.async_copy(rows_v.at[0], o_slice(l0), os0)

            @pl.when(i < L // 2 - 1)
            def _():
                pltpu.async_copy(x_slice(l1 + 1), x_v.at[0], xs0)

            pltpu.make_async_copy(x_slice(l1), x_v.at[1], xs1).wait()

            @pl.when(i >= 1)
            def _():
                pltpu.make_async_copy(rows_v.at[1], o_slice(l1 - 2), os1).wait()

            compute(1, 1)
            pltpu.async_copy(rows_v.at[1], o_slice(l1), os1)
            return carry

        lax.fori_loop(0, L // 2, body2, 0)
        pltpu.make_async_copy(rows_v.at[0], o_slice(L - 2), os0).wait()
        pltpu.make_async_copy(rows_v.at[1], o_slice(L - 1), os1).wait()

    return sc_main


def kernel(x, hour_W, weekday_W, day_W, month_W):
    B, L, _ = x.shape
    D = hour_W.shape[1]
    BH = B // 128
    x = x.astype(jnp.int32)
    W = jnp.concatenate([month_W[:7], day_W[:7], weekday_W[:7], hour_W[:7]], 0)
    T = pl.pallas_call(
        _table_body,
        out_shape=jax.ShapeDtypeStruct((_TR, 128), jnp.float32),
    )(W)
    x3 = (x.transpose(1, 2, 0).reshape(L, 4, BH, 128)
          .transpose(0, 2, 1, 3).reshape(L, BH * 4, 128))
    out4 = _make_sc(B, L, D)(x3, T)
    return (out4.reshape(L, D // 8, BH, 8, 128).transpose(2, 4, 0, 1, 3)
            .reshape(B, L, D))


# speedup vs baseline: 49.0912x; 2.5121x over previous
import functools

import jax
import jax.numpy as jnp
from jax import lax
from jax.experimental import pallas as pl
from jax.experimental.pallas import tpu as pltpu
from jax.experimental.pallas import tpu_sc as plsc

_NC, _NS = 2, 16
_NW = _NC * _NS
_TROWS = 2432


def _table_body(w_ref, t_ref):
    for q in range(4):
        k = 4 * lax.broadcasted_iota(jnp.int32, (_TROWS // 4, 28), 0) + q
        c = lax.broadcasted_iota(jnp.int32, (_TROWS // 4, 28), 1)
        t = c // 7
        v = c % 7
        d0 = k // 343
        d1 = (k // 49) % 7
        d2 = (k // 7) % 7
        d3 = k % 7
        digit = jnp.where(t == 0, d0,
                          jnp.where(t == 1, d1, jnp.where(t == 2, d2, d3)))
        oh = (digit == v).astype(jnp.float32)
        t_ref[:, 32 * q:32 * (q + 1)] = lax.dot_general(
            oh, w_ref[...], (((1,), (0,)), ((), ())),
            precision=lax.Precision.HIGHEST)


def _make_sc(B, L, D):
    BH = B // 128
    NBH = BH // _NW
    DH = D // 8
    mesh = plsc.VectorSubcoreMesh(core_axis_name="c", subcore_axis_name="s",
                                  num_cores=_NC, num_subcores=_NS)

    @functools.partial(
        pl.kernel,
        out_type=jax.ShapeDtypeStruct((L, DH, BH * 8, 128), jnp.float32),
        mesh=mesh,
        compiler_params=pltpu.CompilerParams(needs_layout_passes=False),
        scratch_types=[
            pltpu.VMEM((_TROWS * 33,), jnp.float32),
            pltpu.VMEM((32, 128), jnp.float32),
            pltpu.VMEM((2, NBH * 4, 128), jnp.int32),
            pltpu.VMEM((2, DH, NBH * 8, 128), jnp.float32),
            pltpu.SemaphoreType.DMA,
            pltpu.SemaphoreType.DMA,
            pltpu.SemaphoreType.DMA,
            pltpu.SemaphoreType.DMA,
        ],
    )
    def sc_main(x_hbm, t_hbm, out_hbm, t_v, tmp_v, x_v, rows_v,
                xs0, xs1, os0, os1):
        wid = lax.axis_index("s") * _NC + lax.axis_index("c")
        xcol = wid * (NBH * 4)
        ocol = wid * (NBH * 8)
        lanes = lax.iota(jnp.int32, 16)

        nrow = 32
        for chunk in range(_TROWS // 4 // 32):
            pltpu.sync_copy(t_hbm.at[pl.ds(chunk * nrow, nrow)], tmp_v)

            def repack(r, carry):
                k33 = (chunk * nrow + r) * 4 * 33
                for h in range(8):
                    v = tmp_v[r, pl.ds(h * 16, 16)]
                    dst = (k33 + (h // 2) * 33 + (h % 2) * 16) + lanes
                    plsc.store_scatter(t_v, [dst], v)
                return carry

            lax.fori_loop(0, nrow, repack, 0, unroll=2)

        def x_slice(l):
            return x_hbm.at[l, pl.ds(xcol, NBH * 4)]

        def o_slice(l):
            return out_hbm.at[l, :, pl.ds(ocol, NBH * 8)]

        def compute(xb, rb):
            def group(g, carry2):
                bhi = g // 8
                s = (g % 8) * 16
                x0 = x_v[xb, bhi * 4 + 0, pl.ds(s, 16)]
                x1 = x_v[xb, bhi * 4 + 1, pl.ds(s, 16)]
                x2 = x_v[xb, bhi * 4 + 2, pl.ds(s, 16)]
                x3 = x_v[xb, bhi * 4 + 3, pl.ds(s, 16)]
                key = ((x0 * 7 + x1) * 7 + x2) * 7 + x3
                k33 = key * 33
                for d in range(D):
                    vd = plsc.load_gather(t_v, [k33 + d])
                    rows_v[rb, d // 8, bhi * 8 + d % 8, pl.ds(s, 16)] = vd
                return carry2

            lax.fori_loop(0, NBH * 8, group, 0, unroll=2)

        pltpu.async_copy(x_slice(0), x_v.at[0], xs0)

        def body2(i, carry):
            l0 = 2 * i
            l1 = l0 + 1
            pltpu.async_copy(x_slice(l1), x_v.at[1], xs1)
            pltpu.make_async_copy(x_slice(l0), x_v.at[0], xs0).wait()

            @pl.when(i >= 1)
            def _():
                pltpu.make_async_copy(rows_v.at[0], o_slice(l0 - 2), os0).wait()

            compute(0, 0)
            pltpu.async_copy(rows_v.at[0], o_slice(l0), os0)

            @pl.when(i < L // 2 - 1)
            def _():
                pltpu.async_copy(x_slice(l1 + 1), x_v.at[0], xs0)

            pltpu.make_async_copy(x_slice(l1), x_v.at[1], xs1).wait()

            @pl.when(i >= 1)
            def _():
                pltpu.make_async_copy(rows_v.at[1], o_slice(l1 - 2), os1).wait()

            compute(1, 1)
            pltpu.async_copy(rows_v.at[1], o_slice(l1), os1)
            return carry

        lax.fori_loop(0, L // 2, body2, 0)
        pltpu.make_async_copy(rows_v.at[0], o_slice(L - 2), os0).wait()
        pltpu.make_async_copy(rows_v.at[1], o_slice(L - 1), os1).wait()

    return sc_main


def kernel(x, hour_W, weekday_W, day_W, month_W):
    B, L, _ = x.shape
    D = hour_W.shape[1]
    BH = B // 128
    x = x.astype(jnp.int32)
    W = jnp.concatenate([month_W[:7], day_W[:7], weekday_W[:7], hour_W[:7]], 0)
    T = pl.pallas_call(
        _table_body,
        out_shape=jax.ShapeDtypeStruct((_TROWS // 4, 128), jnp.float32),
    )(W)
    x3 = (x.transpose(1, 2, 0).reshape(L, 4, BH, 128)
          .transpose(0, 2, 1, 3).reshape(L, BH * 4, 128))
    out4 = _make_sc(B, L, D)(x3, T)
    return (out4.reshape(L, D // 8, BH, 8, 128).transpose(2, 4, 0, 1, 3)
            .reshape(B, L, D))


# speedup vs baseline: 96.0806x; 1.9572x over previous
import functools

import jax
import jax.numpy as jnp
from jax import lax
from jax.experimental import pallas as pl
from jax.experimental.pallas import tpu as pltpu
from jax.experimental.pallas import tpu_sc as plsc

_NC, _NS = 2, 16
_NW = _NC * _NS
_TROWS = 2432


def _table_body(w_ref, t_ref):
    for q in range(4):
        k = 4 * lax.broadcasted_iota(jnp.int32, (_TROWS // 4, 28), 0) + q
        c = lax.broadcasted_iota(jnp.int32, (_TROWS // 4, 28), 1)
        t = c // 7
        v = c % 7
        d0 = k // 343
        d1 = (k // 49) % 7
        d2 = (k // 7) % 7
        d3 = k % 7
        digit = jnp.where(t == 0, d0,
                          jnp.where(t == 1, d1, jnp.where(t == 2, d2, d3)))
        oh = (digit == v).astype(jnp.float32)
        t_ref[:, 32 * q:32 * (q + 1)] = lax.dot_general(
            oh, w_ref[...], (((1,), (0,)), ((), ())),
            precision=lax.Precision.HIGHEST)


def _make_sc(B, L, D):
    BH = B // 128
    NBH = BH // _NW
    DH = D // 8
    mesh = plsc.VectorSubcoreMesh(core_axis_name="c", subcore_axis_name="s",
                                  num_cores=_NC, num_subcores=_NS)

    @functools.partial(
        pl.kernel,
        out_type=jax.ShapeDtypeStruct((L, DH, BH * 8, 128), jnp.float32),
        mesh=mesh,
        compiler_params=pltpu.CompilerParams(needs_layout_passes=False),
        scratch_types=[
            pltpu.VMEM((_TROWS * 33,), jnp.float32),
            pltpu.VMEM((32, 128), jnp.float32),
            pltpu.VMEM((2, NBH * 4, 128), jnp.int32),
            pltpu.VMEM((2, DH, NBH * 8, 128), jnp.float32),
            pltpu.SemaphoreType.DMA,
            pltpu.SemaphoreType.DMA,
            pltpu.SemaphoreType.DMA,
            pltpu.SemaphoreType.DMA,
        ],
    )
    def sc_main(x_hbm, t_hbm, out_hbm, t_v, tmp_v, x_v, rows_v,
                xs0, xs1, os0, os1):
        wid = lax.axis_index("s") * _NC + lax.axis_index("c")
        xcol = wid * (NBH * 4)
        ocol = wid * (NBH * 8)
        lanes = lax.iota(jnp.int32, 16)

        nrow = 32
        for chunk in range(_TROWS // 4 // 32):
            pltpu.sync_copy(t_hbm.at[pl.ds(chunk * nrow, nrow)], tmp_v)

            def repack(r, carry):
                k33 = (chunk * nrow + r) * 4 * 33
                for h in range(8):
                    v = tmp_v[r, pl.ds(h * 16, 16)]
                    dst = (k33 + (h // 2) * 33 + (h % 2) * 16) + lanes
                    plsc.store_scatter(t_v, [dst], v)
                return carry

            lax.fori_loop(0, nrow, repack, 0, unroll=2)

        def x_slice(l):
            return x_hbm.at[l, pl.ds(xcol, NBH * 4)]

        def o_slice(l):
            return out_hbm.at[l, :, pl.ds(ocol, NBH * 8)]

        def compute(xb, rb):
            def group(g, carry2):
                bhi = g // 8
                s = (g % 8) * 16
                x0 = x_v[xb, bhi * 4 + 0, pl.ds(s, 16)]
                x1 = x_v[xb, bhi * 4 + 1, pl.ds(s, 16)]
                x2 = x_v[xb, bhi * 4 + 2, pl.ds(s, 16)]
                x3 = x_v[xb, bhi * 4 + 3, pl.ds(s, 16)]
                key = ((x0 * 7 + x1) * 7 + x2) * 7 + x3
                k33 = key * 33
                for d0 in range(0, D, 4):
                    vs = [plsc.load_gather(t_v, [k33 + d])
                          for d in range(d0, d0 + 4)]
                    for j, vd in enumerate(vs):
                        d = d0 + j
                        rows_v[rb, d // 8, bhi * 8 + d % 8, pl.ds(s, 16)] = vd
                return carry2

            lax.fori_loop(0, NBH * 8, group, 0, unroll=2)

        pltpu.async_copy(x_slice(0), x_v.at[0], xs0)

        def body2(i, carry):
            l0 = 2 * i
            l1 = l0 + 1
            pltpu.async_copy(x_slice(l1), x_v.at[1], xs1)
            pltpu.make_async_copy(x_slice(l0), x_v.at[0], xs0).wait()

            @pl.when(i >= 1)
            def _():
                pltpu.make_async_copy(rows_v.at[0], o_slice(l0 - 2), os0).wait()

            compute(0, 0)
            pltpu.async_copy(rows_v.at[0], o_slice(l0), os0)

            @pl.when(i < L // 2 - 1)
            def _():
                pltpu.async_copy(x_slice(l1 + 1), x_v.at[0], xs0)

            pltpu.make_async_copy(x_slice(l1), x_v.at[1], xs1).wait()

            @pl.when(i >= 1)
            def _():
                pltpu.make_async_copy(rows_v.at[1], o_slice(l1 - 2), os1).wait()

            compute(1, 1)
            pltpu.async_copy(rows_v.at[1], o_slice(l1), os1)
            return carry

        lax.fori_loop(0, L // 2, body2, 0)
        pltpu.make_async_copy(rows_v.at[0], o_slice(L - 2), os0).wait()
        pltpu.make_async_copy(rows_v.at[1], o_slice(L - 1), os1).wait()

    return sc_main


def kernel(x, hour_W, weekday_W, day_W, month_W):
    B, L, _ = x.shape
    D = hour_W.shape[1]
    BH = B // 128
    x = x.astype(jnp.int32)
    W = jnp.concatenate([month_W[:7], day_W[:7], weekday_W[:7], hour_W[:7]], 0)
    T = pl.pallas_call(
        _table_body,
        out_shape=jax.ShapeDtypeStruct((_TROWS // 4, 128), jnp.float32),
    )(W)
    x3 = (x.transpose(1, 2, 0).reshape(L, 4, BH, 128)
          .transpose(0, 2, 1, 3).reshape(L, BH * 4, 128))
    out4 = _make_sc(B, L, D)(x3, T)
    return (out4.reshape(L, D // 8, BH, 8, 128).transpose(2, 4, 0, 1, 3)
            .reshape(B, L, D))
